# baseline (speedup 1.0000x reference)
import functools

import jax
import jax.numpy as jnp
from jax import lax
from jax.experimental import pallas as pl
from jax.experimental.pallas import tpu as pltpu
from jax.experimental.pallas import tpu_sc as plsc

_ACTS = ['gelu', 'silu', 'relu', 'leaky_relu']
_DEPTHS = [1, 2, 3]
_HIDS = [1024, 2048, 3072]

_B = 256
_NEB = 8
_ECAP = (_NEB + 1) * _B
_NE = 8
_SLOTS = _NE * _ECAP


def _cfg(i):
    return _ACTS[i % 4], _DEPTHS[i % 3], _HIDS[i % 3]


def _apply_act(name, h):
    if name == 'gelu':
        return 0.5 * h * (1.0 + jax.lax.erf(h * (2.0 ** -0.5)))
    if name == 'silu':
        return h * (1.0 / (1.0 + jnp.exp(-h)))
    if name == 'relu':
        return jnp.maximum(h, 0.0)
    return jnp.where(h >= 0, h, 0.01 * h)


def _layernorm(h, g, b):
    m = jnp.mean(h, axis=-1, keepdims=True)
    v = jnp.mean((h - m) ** 2, axis=-1, keepdims=True)
    return (h - m) / jnp.sqrt(v + 1e-5) * g + b


def _gate_meta_body(x_ref, gw_ref, gb_ref, v_ref, pos_ref, nb_ref):
    logits = jnp.dot(x_ref[...], gw_ref[...],
                     preferred_element_type=jnp.float32) + gb_ref[...]
    m = jnp.max(logits, axis=-1, keepdims=True)
    ex = jnp.exp(logits - m)
    s = ex / jnp.sum(ex, axis=-1, keepdims=True)
    n, e = s.shape
    col = jax.lax.broadcasted_iota(jnp.int32, (n, e), 1)
    v1 = jnp.max(s, axis=-1, keepdims=True)
    i1 = jnp.min(jnp.where(s == v1, col, e), axis=-1, keepdims=True)
    s2 = jnp.where(col == i1, -1.0, s)
    v2 = jnp.max(s2, axis=-1, keepdims=True)
    i2 = jnp.min(jnp.where(s2 == v2, col, e), axis=-1, keepdims=True)
    v_ref[...] = jnp.concatenate([v1, v2], axis=1)

    ch = 1024
    iota8 = jax.lax.broadcasted_iota(jnp.int32, (1, _NE), 1)
    rr = jax.lax.broadcasted_iota(jnp.int32, (ch, ch), 0)
    cc = jax.lax.broadcasted_iota(jnp.int32, (ch, ch), 1)
    tril = (cc < rr).astype(jnp.float32)
    base8 = (iota8 * _ECAP).astype(jnp.float32)
    carry = jnp.zeros((1, _NE), jnp.float32)
    nch = (2 * n) // ch
    for c in range(nch):
        src = i1 if c < nch // 2 else i2
        lo = (c % (nch // 2)) * ch
        ev = src[lo:lo + ch, :]
        oh = (ev == iota8).astype(jnp.float32)
        ranks = jnp.dot(tril, oh, preferred_element_type=jnp.float32) + carry
        posv = jnp.sum(oh * (ranks + base8), axis=1, keepdims=True)
        pos_ref[pl.ds(c * ch, ch), :] = posv.astype(jnp.int32)
        carry = carry + jnp.sum(oh, axis=0, keepdims=True)
    nb_ref[...] = jnp.floor((carry + (_B - 1)) * (1.0 / _B)).astype(jnp.int32)


def _gate_meta(x, gw, gb):
    n = x.shape[0]
    ne = gw.shape[1]
    return pl.pallas_call(
        _gate_meta_body,
        out_shape=[jax.ShapeDtypeStruct((n, 2), jnp.float32),
                   jax.ShapeDtypeStruct((2 * n, 1), jnp.int32),
                   jax.ShapeDtypeStruct((1, _NE), jnp.int32)],
    )(x, gw, gb.reshape(1, ne))


def _sc_dispatch(x, p0, p1):
    n, d = x.shape
    cpt = n // 32
    mesh = plsc.VectorSubcoreMesh(core_axis_name="c", subcore_axis_name="s")

    @functools.partial(
        pl.kernel, mesh=mesh,
        out_type=jax.ShapeDtypeStruct((_SLOTS, d), jnp.float32),
        scratch_types=[
            pltpu.VMEM((cpt,), jnp.int32),
            pltpu.VMEM((cpt,), jnp.int32),
            pltpu.VMEM((cpt, d), jnp.float32),
            pltpu.SemaphoreType.DMA,
        ],
    )
    def k(x_hbm, p0_hbm, p1_hbm, xs_hbm, i0_v, i1_v, rows_v, sem):
        wid = lax.axis_index("s") * 2 + lax.axis_index("c")
        base = wid * cpt
        pltpu.sync_copy(p0_hbm.at[pl.ds(base, cpt)], i0_v)
        pltpu.sync_copy(p1_hbm.at[pl.ds(base, cpt)], i1_v)
        pltpu.sync_copy(x_hbm.at[pl.ds(base, cpt)], rows_v)
        pltpu.async_copy(rows_v, xs_hbm.at[i0_v], sem).wait()
        pltpu.async_copy(rows_v, xs_hbm.at[i1_v], sem).wait()

    return k(x, p0, p1)


def _sc_combine(ys, p0, p1):
    n = p0.shape[0]
    d = ys.shape[1]
    cpt = n // 32
    half = cpt // 2
    mesh = plsc.VectorSubcoreMesh(core_axis_name="c", subcore_axis_name="s")

    @functools.partial(
        pl.kernel, mesh=mesh,
        out_type=(jax.ShapeDtypeStruct((n, d), jnp.float32),
                  jax.ShapeDtypeStruct((n, d), jnp.float32)),
        scratch_types=[
            pltpu.VMEM((half,), jnp.int32),
            pltpu.VMEM((half, d), jnp.float32),
            pltpu.SemaphoreType.DMA,
        ],
    )
    def k(ys_hbm, p0_hbm, p1_hbm, g0_hbm, g1_hbm, i_v, buf_v, sem):
        wid = lax.axis_index("s") * 2 + lax.axis_index("c")
        base = wid * cpt
        for c in range(2):
            b2 = base + c * half
            pltpu.sync_copy(p0_hbm.at[pl.ds(b2, half)], i_v)
            pltpu.async_copy(ys_hbm.at[i_v], buf_v, sem).wait()
            pltpu.sync_copy(buf_v, g0_hbm.at[pl.ds(b2, half)])
            pltpu.sync_copy(p1_hbm.at[pl.ds(b2, half)], i_v)
            pltpu.async_copy(ys_hbm.at[i_v], buf_v, sem).wait()
            pltpu.sync_copy(buf_v, g1_hbm.at[pl.ds(b2, half)])

    return k(ys, p0, p1)


def _ragged_hidden(h_in, W, b, g, beta, act, e, nbf, first):
    K, N = W.shape
    nk = K // 1024
    nbe = nbf[e] + 1

    def xmap(j, k, nb):
        jj = jnp.where(j < nb[e], j, _NEB)
        return (9 * e + jj, k) if first else (jj, k)

    def omap(j, k, nb):
        return (jnp.where(j >= 1, j - 1, _NEB), 0)

    def body(nb_ref, x_ref, w_ref, b_ref, g_ref, bt_ref, o_ref, acc):
        j = pl.program_id(0)
        k = pl.program_id(1)
        jp = j % 2

        @pl.when(j < nb_ref[e])
        def _():
            xv = x_ref[...]
            if first:
                xv = xv.astype(jnp.bfloat16)
            prod = jnp.dot(xv, w_ref[...].astype(jnp.bfloat16),
                           preferred_element_type=jnp.float32)

            @pl.when(k == 0)
            def _():
                acc[jp] = prod

            @pl.when(k > 0)
            def _():
                acc[jp] += prod

        @pl.when((k == 0) & (j >= 1))
        def _():
            hh = acc[1 - jp] + b_ref[...]
            hh = _apply_act(act, hh)
            hh = _layernorm(hh, g_ref[...], bt_ref[...])
            o_ref[...] = hh.astype(jnp.bfloat16)

    grid_spec = pltpu.PrefetchScalarGridSpec(
        num_scalar_prefetch=1,
        grid=(nbe, nk),
        in_specs=[
            pl.BlockSpec((_B, 1024), xmap),
            pl.BlockSpec((1024, N), lambda j, k, nb: (k, 0)),
            pl.BlockSpec((1, N), lambda j, k, nb: (0, 0)),
            pl.BlockSpec((1, N), lambda j, k, nb: (0, 0)),
            pl.BlockSpec((1, N), lambda j, k, nb: (0, 0)),
        ],
        out_specs=pl.BlockSpec((_B, N), omap),
        scratch_shapes=[pltpu.VMEM((2, _B, N), jnp.float32)],
    )
    return pl.pallas_call(
        body,
        grid_spec=grid_spec,
        out_shape=jax.ShapeDtypeStruct((_ECAP, N), jnp.bfloat16),
    )(nbf, h_in, W, b.reshape(1, N), g.reshape(1, N), beta.reshape(1, N))


def _ragged_hidden_final(ys_prev, h_in, l1, l2, act, e, nbf, first):
    W1, W2 = l1['W'], l2['W']
    K, N = W1.shape
    D = W2.shape[1]
    nk = K // 1024
    nbe = nbf[e] + 1

    def xmap(j, k, nb):
        jj = jnp.where(j < nb[e], j, _NEB)
        return (9 * e + jj, k) if first else (jj, k)

    def omap(j, k, nb):
        return (9 * e + jnp.where(j >= 1, j - 1, _NEB), 0)

    def body(nb_ref, *refs):
        if ys_prev is None:
            (x_ref, w1_ref, b1_ref, g1_ref, t1_ref,
             w2_ref, b2_ref, g2_ref, t2_ref, o_ref, acc) = refs
        else:
            (_yp, x_ref, w1_ref, b1_ref, g1_ref, t1_ref,
             w2_ref, b2_ref, g2_ref, t2_ref, o_ref, acc) = refs
        j = pl.program_id(0)
        k = pl.program_id(1)
        jp = j % 2

        @pl.when(j < nb_ref[e])
        def _():
            xv = x_ref[...]
            if first:
                xv = xv.astype(jnp.bfloat16)
            prod = jnp.dot(xv, w1_ref[...].astype(jnp.bfloat16),
                           preferred_element_type=jnp.float32)

            @pl.when(k == 0)
            def _():
                acc[jp] = prod

            @pl.when(k > 0)
            def _():
                acc[jp] += prod

        @pl.when((k == 0) & (j >= 1))
        def _():
            hh = acc[1 - jp] + b1_ref[...]
            hh = _apply_act(act, hh)
            hh = _layernorm(hh, g1_ref[...], t1_ref[...])
            h2 = jnp.dot(hh.astype(jnp.bfloat16),
                         w2_ref[...].astype(jnp.bfloat16),
                         preferred_element_type=jnp.float32) + b2_ref[...]
            o_ref[...] = _layernorm(h2, g2_ref[...], t2_ref[...])

    in_specs = [
        pl.BlockSpec((_B, 1024), xmap),
        pl.BlockSpec((1024, N), lambda j, k, nb: (k, 0)),
        pl.BlockSpec((1, N), lambda j, k, nb: (0, 0)),
        pl.BlockSpec((1, N), lambda j, k, nb: (0, 0)),
        pl.BlockSpec((1, N), lambda j, k, nb: (0, 0)),
        pl.BlockSpec((N, D), lambda j, k, nb: (0, 0)),
        pl.BlockSpec((1, D), lambda j, k, nb: (0, 0)),
        pl.BlockSpec((1, D), lambda j, k, nb: (0, 0)),
        pl.BlockSpec((1, D), lambda j, k, nb: (0, 0)),
    ]
    args = [nbf, h_in, W1, l1['b'].reshape(1, N), l1['g'].reshape(1, N),
            l1['beta'].reshape(1, N), W2, l2['b'].reshape(1, D),
            l2['g'].reshape(1, D), l2['beta'].reshape(1, D)]
    aliases = {}
    if ys_prev is not None:
        in_specs.insert(0, pl.BlockSpec(memory_space=pl.ANY))
        args.insert(1, ys_prev)
        aliases = {1: 0}
    grid_spec = pltpu.PrefetchScalarGridSpec(
        num_scalar_prefetch=1,
        grid=(nbe, nk),
        in_specs=in_specs,
        out_specs=pl.BlockSpec((_B, D), omap),
        scratch_shapes=[pltpu.VMEM((2, _B, N), jnp.float32)],
    )
    return pl.pallas_call(
        body,
        grid_spec=grid_spec,
        out_shape=jax.ShapeDtypeStruct((_SLOTS, D), jnp.float32),
        input_output_aliases=aliases,
    )(*args)


def _mix_body(v_ref, a_ref, b_ref, o_ref):
    v = v_ref[...]
    o_ref[...] = v[:, 0:1] * a_ref[...] + v[:, 1:2] * b_ref[...]


def _mix(val2, g0, g1):
    n, d = g0.shape
    bt = 512
    return pl.pallas_call(
        _mix_body,
        grid=(n // bt,),
        in_specs=[
            pl.BlockSpec((bt, 2), lambda t: (t, 0)),
            pl.BlockSpec((bt, d), lambda t: (t, 0)),
            pl.BlockSpec((bt, d), lambda t: (t, 0)),
        ],
        out_specs=pl.BlockSpec((bt, d), lambda t: (t, 0)),
        out_shape=jax.ShapeDtypeStruct((n, d), jnp.float32),
    )(val2, g0, g1)


def kernel(x, params):
    n = x.shape[0]
    val2, pos, nb8 = _gate_meta(x, params['gate_W'], params['gate_b'])
    p0 = pos[:n].reshape(n)
    p1 = pos[n:].reshape(n)
    nbf = nb8.reshape(_NE)
    xs = _sc_dispatch(x, p0, p1)
    ys = None
    for e in range(_NE):
        act, depth, _hid = _cfg(e)
        L = params['experts'][e]
        if depth == 1:
            ys = _ragged_hidden_final(ys, xs, L[0], L[1], act, e, nbf,
                                      first=True)
        elif depth == 2:
            h = _ragged_hidden(xs, L[0]['W'], L[0]['b'], L[0]['g'],
                               L[0]['beta'], act, e, nbf, first=True)
            ys = _ragged_hidden_final(ys, h, L[1], L[2], act, e, nbf,
                                      first=False)
        else:
            h = _ragged_hidden(xs, L[0]['W'], L[0]['b'], L[0]['g'],
                               L[0]['beta'], act, e, nbf, first=True)
            h = _ragged_hidden(h, L[1]['W'], L[1]['b'], L[1]['g'],
                               L[1]['beta'], act, e, nbf, first=False)
            ys = _ragged_hidden_final(ys, h, L[2], L[3], act, e, nbf,
                                      first=False)
    g0, g1 = _sc_combine(ys, p0, p1)
    return _mix(val2, g0, g1)


# speedup vs baseline: 1.1989x; 1.1989x over previous
import functools

import jax
import jax.numpy as jnp
from jax import lax
from jax.experimental import pallas as pl
from jax.experimental.pallas import tpu as pltpu
from jax.experimental.pallas import tpu_sc as plsc

_ACTS = ['gelu', 'silu', 'relu', 'leaky_relu']
_DEPTHS = [1, 2, 3]
_HIDS = [1024, 2048, 3072]

_B = 256
_NEB = 8
_ECAP = (_NEB + 1) * _B
_NE = 8
_SLOTS = _NE * _ECAP


def _cfg(i):
    return _ACTS[i % 4], _DEPTHS[i % 3], _HIDS[i % 3]


def _apply_act(name, h):
    if name == 'gelu':
        return 0.5 * h * (1.0 + jax.lax.erf(h * (2.0 ** -0.5)))
    if name == 'silu':
        return h * (1.0 / (1.0 + jnp.exp(-h)))
    if name == 'relu':
        return jnp.maximum(h, 0.0)
    return jnp.where(h >= 0, h, 0.01 * h)


def _layernorm(h, g, b):
    m = jnp.mean(h, axis=-1, keepdims=True)
    v = jnp.mean((h - m) ** 2, axis=-1, keepdims=True)
    return (h - m) / jnp.sqrt(v + 1e-5) * g + b


def _act_ln(act, acc, b, g, t):
    hh = _apply_act(act, acc + b)
    return _layernorm(hh, g, t)


def _c3(v, n):
    return [pl.BlockSpec((1, n), lambda j, k, nb: (0, 0))] * 3, \
           [v['b'].reshape(1, n), v['g'].reshape(1, n), v['beta'].reshape(1, n)]


def _gate_meta_body(x_ref, gw_ref, gb_ref, v_ref, pos_ref, nb_ref):
    logits = jnp.dot(x_ref[...], gw_ref[...],
                     preferred_element_type=jnp.float32) + gb_ref[...]
    m = jnp.max(logits, axis=-1, keepdims=True)
    ex = jnp.exp(logits - m)
    s = ex / jnp.sum(ex, axis=-1, keepdims=True)
    n, e = s.shape
    col = jax.lax.broadcasted_iota(jnp.int32, (n, e), 1)
    v1 = jnp.max(s, axis=-1, keepdims=True)
    i1 = jnp.min(jnp.where(s == v1, col, e), axis=-1, keepdims=True)
    s2 = jnp.where(col == i1, -1.0, s)
    v2 = jnp.max(s2, axis=-1, keepdims=True)
    i2 = jnp.min(jnp.where(s2 == v2, col, e), axis=-1, keepdims=True)
    v_ref[...] = jnp.concatenate([v1, v2], axis=1)

    ch = 1024
    iota8 = jax.lax.broadcasted_iota(jnp.int32, (1, _NE), 1)
    rr = jax.lax.broadcasted_iota(jnp.int32, (ch, ch), 0)
    cc = jax.lax.broadcasted_iota(jnp.int32, (ch, ch), 1)
    tril = (cc < rr).astype(jnp.float32)
    base8 = (iota8 * _ECAP).astype(jnp.float32)
    carry = jnp.zeros((1, _NE), jnp.float32)
    nch = (2 * n) // ch
    for c in range(nch):
        src = i1 if c < nch // 2 else i2
        lo = (c % (nch // 2)) * ch
        ev = src[lo:lo + ch, :]
        oh = (ev == iota8).astype(jnp.float32)
        ranks = jnp.dot(tril, oh, preferred_element_type=jnp.float32) + carry
        posv = jnp.sum(oh * (ranks + base8), axis=1, keepdims=True)
        pos_ref[pl.ds(c * ch, ch), :] = posv.astype(jnp.int32)
        carry = carry + jnp.sum(oh, axis=0, keepdims=True)
    nb_ref[...] = jnp.floor((carry + (_B - 1)) * (1.0 / _B)).astype(jnp.int32)


def _gate_meta(x, gw, gb):
    n = x.shape[0]
    ne = gw.shape[1]
    return pl.pallas_call(
        _gate_meta_body,
        out_shape=[jax.ShapeDtypeStruct((n, 2), jnp.float32),
                   jax.ShapeDtypeStruct((2 * n, 1), jnp.int32),
                   jax.ShapeDtypeStruct((1, _NE), jnp.int32)],
    )(x, gw, gb.reshape(1, ne))


def _sc_dispatch(x, p0, p1):
    n, d = x.shape
    cpt = n // 32
    mesh = plsc.VectorSubcoreMesh(core_axis_name="c", subcore_axis_name="s")

    @functools.partial(
        pl.kernel, mesh=mesh,
        out_type=jax.ShapeDtypeStruct((_SLOTS, d), jnp.float32),
        scratch_types=[
            pltpu.VMEM((cpt,), jnp.int32),
            pltpu.VMEM((cpt,), jnp.int32),
            pltpu.VMEM((cpt, d), jnp.float32),
            pltpu.SemaphoreType.DMA,
        ],
    )
    def k(x_hbm, p0_hbm, p1_hbm, xs_hbm, i0_v, i1_v, rows_v, sem):
        wid = lax.axis_index("s") * 2 + lax.axis_index("c")
        base = wid * cpt
        pltpu.sync_copy(p0_hbm.at[pl.ds(base, cpt)], i0_v)
        pltpu.sync_copy(p1_hbm.at[pl.ds(base, cpt)], i1_v)
        pltpu.sync_copy(x_hbm.at[pl.ds(base, cpt)], rows_v)
        pltpu.async_copy(rows_v, xs_hbm.at[i0_v], sem).wait()
        pltpu.async_copy(rows_v, xs_hbm.at[i1_v], sem).wait()

    return k(x, p0, p1)


def _sc_combine(ys, p0, p1):
    n = p0.shape[0]
    d = ys.shape[1]
    cpt = n // 32
    half = cpt // 2
    mesh = plsc.VectorSubcoreMesh(core_axis_name="c", subcore_axis_name="s")

    @functools.partial(
        pl.kernel, mesh=mesh,
        out_type=(jax.ShapeDtypeStruct((n, d), jnp.float32),
                  jax.ShapeDtypeStruct((n, d), jnp.float32)),
        scratch_types=[
            pltpu.VMEM((half,), jnp.int32),
            pltpu.VMEM((half, d), jnp.float32),
            pltpu.SemaphoreType.DMA,
        ],
    )
    def k(ys_hbm, p0_hbm, p1_hbm, g0_hbm, g1_hbm, i_v, buf_v, sem):
        wid = lax.axis_index("s") * 2 + lax.axis_index("c")
        base = wid * cpt
        for c in range(2):
            b2 = base + c * half
            pltpu.sync_copy(p0_hbm.at[pl.ds(b2, half)], i_v)
            pltpu.async_copy(ys_hbm.at[i_v], buf_v, sem).wait()
            pltpu.sync_copy(buf_v, g0_hbm.at[pl.ds(b2, half)])
            pltpu.sync_copy(p1_hbm.at[pl.ds(b2, half)], i_v)
            pltpu.async_copy(ys_hbm.at[i_v], buf_v, sem).wait()
            pltpu.sync_copy(buf_v, g1_hbm.at[pl.ds(b2, half)])

    return k(ys, p0, p1)


def _ragged_hidden(h_in, W, b, g, beta, act, e, nbf, first):
    K, N = W.shape
    nk = K // 1024
    nbe = jnp.maximum(nbf[e], 1)

    def xmap(j, k, nb):
        jj = jnp.where(j < nb[e], j, _NEB)
        return (9 * e + jj, k) if first else (jj, k)

    def omap(j, k, nb):
        return (jnp.where(j < nb[e], j, _NEB), 0)

    def body(nb_ref, x_ref, w_ref, b_ref, g_ref, t_ref, o_ref, acc):
        k = pl.program_id(1)
        xv = x_ref[...]
        if first:
            xv = xv.astype(jnp.bfloat16)
        prod = jnp.dot(xv, w_ref[...].astype(jnp.bfloat16),
                       preferred_element_type=jnp.float32)

        @pl.when(k == 0)
        def _():
            acc[...] = prod

        @pl.when(k > 0)
        def _():
            acc[...] += prod

        @pl.when(k == nk - 1)
        def _():
            o_ref[...] = _act_ln(act, acc[...], b_ref[...], g_ref[...],
                                 t_ref[...]).astype(jnp.bfloat16)

    c3, cargs = _c3({'b': b, 'g': g, 'beta': beta}, N)
    grid_spec = pltpu.PrefetchScalarGridSpec(
        num_scalar_prefetch=1,
        grid=(nbe, nk),
        in_specs=[
            pl.BlockSpec((_B, 1024), xmap),
            pl.BlockSpec((1024, N), lambda j, k, nb: (k, 0)),
        ] + c3,
        out_specs=pl.BlockSpec((_B, N), omap),
        scratch_shapes=[pltpu.VMEM((_B, N), jnp.float32)],
    )
    return pl.pallas_call(
        body,
        grid_spec=grid_spec,
        out_shape=jax.ShapeDtypeStruct((_ECAP, N), jnp.bfloat16),
    )(nbf, h_in, W, *cargs)


def _ragged_c12(xs, l1, l2, act, e, nbf):
    W1, W2 = l1['W'], l2['W']
    N = W1.shape[1]
    nk = N // 1024
    nbe = jnp.maximum(nbf[e], 1)

    def xmap(j, k, nb):
        return (9 * e + jnp.where(j < nb[e], j, _NEB), 0)

    def omap(j, k, nb):
        return (jnp.where(j < nb[e], j, _NEB), 0)

    def body(nb_ref, x_ref, w1_ref, b1_ref, g1_ref, t1_ref,
             w2_ref, b2_ref, g2_ref, t2_ref, o_ref, acc, h1s):
        k = pl.program_id(1)

        @pl.when(k == 0)
        def _():
            h1 = jnp.dot(x_ref[...].astype(jnp.bfloat16),
                         w1_ref[...].astype(jnp.bfloat16),
                         preferred_element_type=jnp.float32)
            h1v = _act_ln(act, h1, b1_ref[...], g1_ref[...],
                          t1_ref[...]).astype(jnp.bfloat16)
            for i in range(nk):
                h1s[i] = h1v[:, i * 1024:(i + 1) * 1024]

        prod = jnp.dot(h1s[k],
                       w2_ref[...].astype(jnp.bfloat16),
                       preferred_element_type=jnp.float32)

        @pl.when(k == 0)
        def _():
            acc[...] = prod

        @pl.when(k > 0)
        def _():
            acc[...] += prod

        @pl.when(k == nk - 1)
        def _():
            o_ref[...] = _act_ln(act, acc[...], b2_ref[...], g2_ref[...],
                                 t2_ref[...]).astype(jnp.bfloat16)

    c31, a1 = _c3(l1, N)
    c32, a2 = _c3(l2, N)
    grid_spec = pltpu.PrefetchScalarGridSpec(
        num_scalar_prefetch=1,
        grid=(nbe, nk),
        in_specs=[
            pl.BlockSpec((_B, 1024), xmap),
            pl.BlockSpec((1024, N), lambda j, k, nb: (0, 0)),
        ] + c31 + [
            pl.BlockSpec((1024, N), lambda j, k, nb: (k, 0)),
        ] + c32,
        out_specs=pl.BlockSpec((_B, N), omap),
        scratch_shapes=[pltpu.VMEM((_B, N), jnp.float32),
                        pltpu.VMEM((nk, _B, 1024), jnp.bfloat16)],
    )
    return pl.pallas_call(
        body,
        grid_spec=grid_spec,
        out_shape=jax.ShapeDtypeStruct((_ECAP, N), jnp.bfloat16),
    )(nbf, xs, W1, *a1, W2, *a2)


def _ragged_b_full(ys_prev, xs, l1, l2, l3, act, e, nbf):
    W1, W2, W3 = l1['W'], l2['W'], l3['W']
    N = W1.shape[1]
    D = W3.shape[1]
    nk = N // 1024
    nbe = jnp.maximum(nbf[e], 1)

    def xmap(j, k, nb):
        return (9 * e + jnp.where(j < nb[e], j, _NEB), 0)

    def omap(j, k, nb):
        return (9 * e + jnp.where(j < nb[e], j, _NEB), 0)

    def body(nb_ref, *refs):
        (_yp, x_ref, w1_ref, b1_ref, g1_ref, t1_ref,
         w2_ref, b2_ref, g2_ref, t2_ref,
         w3_ref, b3_ref, g3_ref, t3_ref, o_ref, acc, h1s) = refs
        k = pl.program_id(1)

        @pl.when(k == 0)
        def _():
            h1 = jnp.dot(x_ref[...].astype(jnp.bfloat16),
                         w1_ref[...].astype(jnp.bfloat16),
                         preferred_element_type=jnp.float32)
            h1v = _act_ln(act, h1, b1_ref[...], g1_ref[...],
                          t1_ref[...]).astype(jnp.bfloat16)
            for i in range(nk):
                h1s[i] = h1v[:, i * 1024:(i + 1) * 1024]

        prod = jnp.dot(h1s[k],
                       w2_ref[...].astype(jnp.bfloat16),
                       preferred_element_type=jnp.float32)

        @pl.when(k == 0)
        def _():
            acc[...] = prod

        @pl.when(k > 0)
        def _():
            acc[...] += prod

        @pl.when(k == nk - 1)
        def _():
            h2 = _act_ln(act, acc[...], b2_ref[...], g2_ref[...], t2_ref[...])
            h3 = jnp.dot(h2.astype(jnp.bfloat16),
                         w3_ref[...].astype(jnp.bfloat16),
                         preferred_element_type=jnp.float32) + b3_ref[...]
            o_ref[...] = _layernorm(h3, g3_ref[...], t3_ref[...])

    c31, a1 = _c3(l1, N)
    c32, a2 = _c3(l2, N)
    c33, a3 = _c3(l3, D)
    in_specs = [
        pl.BlockSpec(memory_space=pl.ANY),
        pl.BlockSpec((_B, 1024), xmap),
        pl.BlockSpec((1024, N), lambda j, k, nb: (0, 0)),
    ] + c31 + [
        pl.BlockSpec((1024, N), lambda j, k, nb: (k, 0)),
    ] + c32 + [
        pl.BlockSpec((N, D), lambda j, k, nb: (0, 0)),
    ] + c33
    grid_spec = pltpu.PrefetchScalarGridSpec(
        num_scalar_prefetch=1,
        grid=(nbe, nk),
        in_specs=in_specs,
        out_specs=pl.BlockSpec((_B, D), omap),
        scratch_shapes=[pltpu.VMEM((_B, N), jnp.float32),
                        pltpu.VMEM((nk, _B, 1024), jnp.bfloat16)],
    )
    return pl.pallas_call(
        body,
        grid_spec=grid_spec,
        out_shape=jax.ShapeDtypeStruct((_SLOTS, D), jnp.float32),
        input_output_aliases={1: 0},
    )(nbf, ys_prev, xs, W1, *a1, W2, *a2, W3, *a3)


def _ragged_hidden_final(ys_prev, h_in, l1, l2, act, e, nbf, first):
    W1, W2 = l1['W'], l2['W']
    K, N = W1.shape
    D = W2.shape[1]
    nk = K // 1024
    nbe = jnp.maximum(nbf[e], 1)

    def xmap(j, k, nb):
        jj = jnp.where(j < nb[e], j, _NEB)
        return (9 * e + jj, k) if first else (jj, k)

    def omap(j, k, nb):
        return (9 * e + jnp.where(j < nb[e], j, _NEB), 0)

    def body(nb_ref, *refs):
        if ys_prev is None:
            (x_ref, w1_ref, b1_ref, g1_ref, t1_ref,
             w2_ref, b2_ref, g2_ref, t2_ref, o_ref, acc) = refs
        else:
            (_yp, x_ref, w1_ref, b1_ref, g1_ref, t1_ref,
             w2_ref, b2_ref, g2_ref, t2_ref, o_ref, acc) = refs
        k = pl.program_id(1)
        xv = x_ref[...]
        if first:
            xv = xv.astype(jnp.bfloat16)
        prod = jnp.dot(xv, w1_ref[...].astype(jnp.bfloat16),
                       preferred_element_type=jnp.float32)

        @pl.when(k == 0)
        def _():
            acc[...] = prod

        @pl.when(k > 0)
        def _():
            acc[...] += prod

        @pl.when(k == nk - 1)
        def _():
            hh = _act_ln(act, acc[...], b1_ref[...], g1_ref[...], t1_ref[...])
            h2 = jnp.dot(hh.astype(jnp.bfloat16),
                         w2_ref[...].astype(jnp.bfloat16),
                         preferred_element_type=jnp.float32) + b2_ref[...]
            o_ref[...] = _layernorm(h2, g2_ref[...], t2_ref[...])

    c31, a1 = _c3(l1, N)
    c32, a2 = _c3(l2, D)
    in_specs = [
        pl.BlockSpec((_B, 1024), xmap),
        pl.BlockSpec((1024, N), lambda j, k, nb: (k, 0)),
    ] + c31 + [
        pl.BlockSpec((N, D), lambda j, k, nb: (0, 0)),
    ] + c32
    args = [nbf, h_in, W1] + a1 + [W2] + a2
    aliases = {}
    if ys_prev is not None:
        in_specs.insert(0, pl.BlockSpec(memory_space=pl.ANY))
        args.insert(1, ys_prev)
        aliases = {1: 0}
    grid_spec = pltpu.PrefetchScalarGridSpec(
        num_scalar_prefetch=1,
        grid=(nbe, nk),
        in_specs=in_specs,
        out_specs=pl.BlockSpec((_B, D), omap),
        scratch_shapes=[pltpu.VMEM((_B, N), jnp.float32)],
    )
    return pl.pallas_call(
        body,
        grid_spec=grid_spec,
        out_shape=jax.ShapeDtypeStruct((_SLOTS, D), jnp.float32),
        input_output_aliases=aliases,
    )(*args)


def _mix_body(v_ref, a_ref, b_ref, o_ref):
    v = v_ref[...]
    o_ref[...] = v[:, 0:1] * a_ref[...] + v[:, 1:2] * b_ref[...]


def _mix(val2, g0, g1):
    n, d = g0.shape
    bt = 512
    return pl.pallas_call(
        _mix_body,
        grid=(n // bt,),
        in_specs=[
            pl.BlockSpec((bt, 2), lambda t: (t, 0)),
            pl.BlockSpec((bt, d), lambda t: (t, 0)),
            pl.BlockSpec((bt, d), lambda t: (t, 0)),
        ],
        out_specs=pl.BlockSpec((bt, d), lambda t: (t, 0)),
        out_shape=jax.ShapeDtypeStruct((n, d), jnp.float32),
    )(val2, g0, g1)


def kernel(x, params):
    n = x.shape[0]
    val2, pos, nb8 = _gate_meta(x, params['gate_W'], params['gate_b'])
    p0 = pos[:n].reshape(n)
    p1 = pos[n:].reshape(n)
    nbf = nb8.reshape(_NE)
    xs = _sc_dispatch(x, p0, p1)
    ys = None
    for e in range(_NE):
        act, depth, _hid = _cfg(e)
        L = params['experts'][e]
        if depth == 1:
            ys = _ragged_hidden_final(ys, xs, L[0], L[1], act, e, nbf,
                                      first=True)
        elif depth == 2:
            ys = _ragged_b_full(ys, xs, L[0], L[1], L[2], act, e, nbf)
        else:
            h = _ragged_c12(xs, L[0], L[1], act, e, nbf)
            ys = _ragged_hidden_final(ys, h, L[2], L[3], act, e, nbf,
                                      first=False)
    g0, g1 = _sc_combine(ys, p0, p1)
    return _mix(val2, g0, g1)


# speedup vs baseline: 1.2047x; 1.0048x over previous
import functools

import jax
import jax.numpy as jnp
from jax import lax
from jax.experimental import pallas as pl
from jax.experimental.pallas import tpu as pltpu
from jax.experimental.pallas import tpu_sc as plsc

_ACTS = ['gelu', 'silu', 'relu', 'leaky_relu']
_DEPTHS = [1, 2, 3]
_HIDS = [1024, 2048, 3072]

_B = 256
_NEB = 8
_ECAP = (_NEB + 1) * _B
_NE = 8
_SLOTS = _NE * _ECAP


def _cfg(i):
    return _ACTS[i % 4], _DEPTHS[i % 3], _HIDS[i % 3]


def _apply_act(name, h):
    if name == 'gelu':
        return 0.5 * h * (1.0 + jax.lax.erf(h * (2.0 ** -0.5)))
    if name == 'silu':
        return h * (1.0 / (1.0 + jnp.exp(-h)))
    if name == 'relu':
        return jnp.maximum(h, 0.0)
    return jnp.where(h >= 0, h, 0.01 * h)


def _layernorm(h, g, b):
    m = jnp.mean(h, axis=-1, keepdims=True)
    v = jnp.mean((h - m) ** 2, axis=-1, keepdims=True)
    return (h - m) / jnp.sqrt(v + 1e-5) * g + b


def _act_ln(act, acc, b, g, t):
    hh = _apply_act(act, acc + b)
    return _layernorm(hh, g, t)


def _c3(v, n):
    return [pl.BlockSpec((1, n), lambda j, k, nb: (0, 0))] * 3, \
           [v['b'].reshape(1, n), v['g'].reshape(1, n), v['beta'].reshape(1, n)]


def _gate_meta_body(x_ref, gw_ref, gb_ref, v_ref, pos_ref, nb_ref):
    logits = jnp.dot(x_ref[...], gw_ref[...],
                     preferred_element_type=jnp.float32) + gb_ref[...]
    m = jnp.max(logits, axis=-1, keepdims=True)
    ex = jnp.exp(logits - m)
    s = ex / jnp.sum(ex, axis=-1, keepdims=True)
    n, e = s.shape
    col = jax.lax.broadcasted_iota(jnp.int32, (n, e), 1)
    v1 = jnp.max(s, axis=-1, keepdims=True)
    i1 = jnp.min(jnp.where(s == v1, col, e), axis=-1, keepdims=True)
    s2 = jnp.where(col == i1, -1.0, s)
    v2 = jnp.max(s2, axis=-1, keepdims=True)
    i2 = jnp.min(jnp.where(s2 == v2, col, e), axis=-1, keepdims=True)
    v_ref[...] = jnp.concatenate([v1, v2], axis=1)

    ch = 1024
    iota8 = jax.lax.broadcasted_iota(jnp.int32, (1, _NE), 1)
    rr = jax.lax.broadcasted_iota(jnp.int32, (ch, ch), 0)
    cc = jax.lax.broadcasted_iota(jnp.int32, (ch, ch), 1)
    tril = (cc < rr).astype(jnp.float32)
    base8 = (iota8 * _ECAP).astype(jnp.float32)
    carry = jnp.zeros((1, _NE), jnp.float32)
    nch = (2 * n) // ch
    for c in range(nch):
        src = i1 if c < nch // 2 else i2
        lo = (c % (nch // 2)) * ch
        ev = src[lo:lo + ch, :]
        oh = (ev == iota8).astype(jnp.float32)
        ranks = jnp.dot(tril, oh, preferred_element_type=jnp.float32) + carry
        posv = jnp.sum(oh * (ranks + base8), axis=1, keepdims=True)
        pos_ref[pl.ds(c * ch, ch), :] = posv.astype(jnp.int32)
        carry = carry + jnp.sum(oh, axis=0, keepdims=True)
    nb_ref[...] = jnp.floor((carry + (_B - 1)) * (1.0 / _B)).astype(jnp.int32)


def _gate_meta(x, gw, gb):
    n = x.shape[0]
    ne = gw.shape[1]
    return pl.pallas_call(
        _gate_meta_body,
        out_shape=[jax.ShapeDtypeStruct((n, 2), jnp.float32),
                   jax.ShapeDtypeStruct((2 * n, 1), jnp.int32),
                   jax.ShapeDtypeStruct((1, _NE), jnp.int32)],
    )(x, gw, gb.reshape(1, ne))


def _sc_dispatch(x, p0, p1):
    n, d = x.shape
    cpt = n // 32
    mesh = plsc.VectorSubcoreMesh(core_axis_name="c", subcore_axis_name="s")

    @functools.partial(
        pl.kernel, mesh=mesh,
        out_type=jax.ShapeDtypeStruct((_SLOTS, d), jnp.float32),
        scratch_types=[
            pltpu.VMEM((cpt,), jnp.int32),
            pltpu.VMEM((cpt,), jnp.int32),
            pltpu.VMEM((cpt, d), jnp.float32),
            pltpu.SemaphoreType.DMA,
            pltpu.SemaphoreType.DMA,
        ],
    )
    def k(x_hbm, p0_hbm, p1_hbm, xs_hbm, i0_v, i1_v, rows_v, sem0, sem1):
        wid = lax.axis_index("s") * 2 + lax.axis_index("c")
        base = wid * cpt
        pltpu.sync_copy(p0_hbm.at[pl.ds(base, cpt)], i0_v)
        pltpu.sync_copy(p1_hbm.at[pl.ds(base, cpt)], i1_v)
        pltpu.sync_copy(x_hbm.at[pl.ds(base, cpt)], rows_v)
        c0 = pltpu.async_copy(rows_v, xs_hbm.at[i0_v], sem0)
        c1 = pltpu.async_copy(rows_v, xs_hbm.at[i1_v], sem1)
        c0.wait()
        c1.wait()

    return k(x, p0, p1)


def _sc_combine(ys, p0, p1):
    n = p0.shape[0]
    d = ys.shape[1]
    cpt = n // 32
    half = cpt // 2
    mesh = plsc.VectorSubcoreMesh(core_axis_name="c", subcore_axis_name="s")

    @functools.partial(
        pl.kernel, mesh=mesh,
        out_type=(jax.ShapeDtypeStruct((n, d), jnp.float32),
                  jax.ShapeDtypeStruct((n, d), jnp.float32)),
        scratch_types=[
            pltpu.VMEM((half,), jnp.int32),
            pltpu.VMEM((half,), jnp.int32),
            pltpu.VMEM((half, d), jnp.float32),
            pltpu.VMEM((half, d), jnp.float32),
            pltpu.SemaphoreType.DMA,
            pltpu.SemaphoreType.DMA,
        ],
    )
    def k(ys_hbm, p0_hbm, p1_hbm, g0_hbm, g1_hbm, i0_v, i1_v,
          buf0_v, buf1_v, sem0, sem1):
        wid = lax.axis_index("s") * 2 + lax.axis_index("c")
        base = wid * cpt
        for c in range(2):
            b2 = base + c * half
            pltpu.sync_copy(p0_hbm.at[pl.ds(b2, half)], i0_v)
            pltpu.sync_copy(p1_hbm.at[pl.ds(b2, half)], i1_v)
            c0 = pltpu.async_copy(ys_hbm.at[i0_v], buf0_v, sem0)
            c1 = pltpu.async_copy(ys_hbm.at[i1_v], buf1_v, sem1)
            c0.wait()
            pltpu.sync_copy(buf0_v, g0_hbm.at[pl.ds(b2, half)])
            c1.wait()
            pltpu.sync_copy(buf1_v, g1_hbm.at[pl.ds(b2, half)])

    return k(ys, p0, p1)


def _ragged_hidden(h_in, W, b, g, beta, act, e, nbf, first):
    K, N = W.shape
    nk = K // 1024
    nbe = jnp.maximum(nbf[e], 1)

    def xmap(j, k, nb):
        jj = jnp.where(j < nb[e], j, _NEB)
        return ((_NEB + 1) * e + jj, k) if first else (jj, k)

    def omap(j, k, nb):
        return (jnp.where(j < nb[e], j, _NEB), 0)

    def body(nb_ref, x_ref, w_ref, b_ref, g_ref, t_ref, o_ref, acc):
        k = pl.program_id(1)
        xv = x_ref[...]
        if first:
            xv = xv.astype(jnp.bfloat16)
        prod = jnp.dot(xv, w_ref[...].astype(jnp.bfloat16),
                       preferred_element_type=jnp.float32)

        @pl.when(k == 0)
        def _():
            acc[...] = prod

        @pl.when(k > 0)
        def _():
            acc[...] += prod

        @pl.when(k == nk - 1)
        def _():
            o_ref[...] = _act_ln(act, acc[...], b_ref[...], g_ref[...],
                                 t_ref[...]).astype(jnp.bfloat16)

    c3, cargs = _c3({'b': b, 'g': g, 'beta': beta}, N)
    grid_spec = pltpu.PrefetchScalarGridSpec(
        num_scalar_prefetch=1,
        grid=(nbe, nk),
        in_specs=[
            pl.BlockSpec((_B, 1024), xmap),
            pl.BlockSpec((1024, N), lambda j, k, nb: (k, 0)),
        ] + c3,
        out_specs=pl.BlockSpec((_B, N), omap),
        scratch_shapes=[pltpu.VMEM((_B, N), jnp.float32)],
    )
    return pl.pallas_call(
        body,
        grid_spec=grid_spec,
        out_shape=jax.ShapeDtypeStruct((_ECAP, N), jnp.bfloat16),
    )(nbf, h_in, W, *cargs)


def _ragged_c12(xs, l1, l2, act, e, nbf):
    W1, W2 = l1['W'], l2['W']
    N = W1.shape[1]
    nk = N // 1024
    nbe = jnp.maximum(nbf[e], 1)

    def xmap(j, k, nb):
        return ((_NEB + 1) * e + jnp.where(j < nb[e], j, _NEB), 0)

    def omap(j, k, nb):
        return (jnp.where(j < nb[e], j, _NEB), 0)

    def body(nb_ref, x_ref, w1_ref, b1_ref, g1_ref, t1_ref,
             w2_ref, b2_ref, g2_ref, t2_ref, o_ref, acc, h1s):
        k = pl.program_id(1)

        @pl.when(k == 0)
        def _():
            h1 = jnp.dot(x_ref[...].astype(jnp.bfloat16),
                         w1_ref[...].astype(jnp.bfloat16),
                         preferred_element_type=jnp.float32)
            h1v = _act_ln(act, h1, b1_ref[...], g1_ref[...],
                          t1_ref[...]).astype(jnp.bfloat16)
            for i in range(nk):
                h1s[i] = h1v[:, i * 1024:(i + 1) * 1024]

        prod = jnp.dot(h1s[k],
                       w2_ref[...].astype(jnp.bfloat16),
                       preferred_element_type=jnp.float32)

        @pl.when(k == 0)
        def _():
            acc[...] = prod

        @pl.when(k > 0)
        def _():
            acc[...] += prod

        @pl.when(k == nk - 1)
        def _():
            o_ref[...] = _act_ln(act, acc[...], b2_ref[...], g2_ref[...],
                                 t2_ref[...]).astype(jnp.bfloat16)

    c31, a1 = _c3(l1, N)
    c32, a2 = _c3(l2, N)
    grid_spec = pltpu.PrefetchScalarGridSpec(
        num_scalar_prefetch=1,
        grid=(nbe, nk),
        in_specs=[
            pl.BlockSpec((_B, 1024), xmap),
            pl.BlockSpec((1024, N), lambda j, k, nb: (0, 0)),
        ] + c31 + [
            pl.BlockSpec((1024, N), lambda j, k, nb: (k, 0)),
        ] + c32,
        out_specs=pl.BlockSpec((_B, N), omap),
        scratch_shapes=[pltpu.VMEM((_B, N), jnp.float32),
                        pltpu.VMEM((nk, _B, 1024), jnp.bfloat16)],
    )
    return pl.pallas_call(
        body,
        grid_spec=grid_spec,
        out_shape=jax.ShapeDtypeStruct((_ECAP, N), jnp.bfloat16),
    )(nbf, xs, W1, *a1, W2, *a2)


def _ragged_b_full(ys_prev, xs, l1, l2, l3, act, e, nbf):
    W1, W2, W3 = l1['W'], l2['W'], l3['W']
    N = W1.shape[1]
    D = W3.shape[1]
    nk = N // 1024
    nbe = jnp.maximum(nbf[e], 1)

    def xmap(j, k, nb):
        return ((_NEB + 1) * e + jnp.where(j < nb[e], j, _NEB), 0)

    def omap(j, k, nb):
        return ((_NEB + 1) * e + jnp.where(j < nb[e], j, _NEB), 0)

    def body(nb_ref, *refs):
        (_yp, x_ref, w1_ref, b1_ref, g1_ref, t1_ref,
         w2_ref, b2_ref, g2_ref, t2_ref,
         w3_ref, b3_ref, g3_ref, t3_ref, o_ref, acc, h1s) = refs
        k = pl.program_id(1)

        @pl.when(k == 0)
        def _():
            h1 = jnp.dot(x_ref[...].astype(jnp.bfloat16),
                         w1_ref[...].astype(jnp.bfloat16),
                         preferred_element_type=jnp.float32)
            h1v = _act_ln(act, h1, b1_ref[...], g1_ref[...],
                          t1_ref[...]).astype(jnp.bfloat16)
            for i in range(nk):
                h1s[i] = h1v[:, i * 1024:(i + 1) * 1024]

        prod = jnp.dot(h1s[k],
                       w2_ref[...].astype(jnp.bfloat16),
                       preferred_element_type=jnp.float32)

        @pl.when(k == 0)
        def _():
            acc[...] = prod

        @pl.when(k > 0)
        def _():
            acc[...] += prod

        @pl.when(k == nk - 1)
        def _():
            h2 = _act_ln(act, acc[...], b2_ref[...], g2_ref[...], t2_ref[...])
            h3 = jnp.dot(h2.astype(jnp.bfloat16),
                         w3_ref[...].astype(jnp.bfloat16),
                         preferred_element_type=jnp.float32) + b3_ref[...]
            o_ref[...] = _layernorm(h3, g3_ref[...], t3_ref[...])

    c31, a1 = _c3(l1, N)
    c32, a2 = _c3(l2, N)
    c33, a3 = _c3(l3, D)
    in_specs = [
        pl.BlockSpec(memory_space=pl.ANY),
        pl.BlockSpec((_B, 1024), xmap),
        pl.BlockSpec((1024, N), lambda j, k, nb: (0, 0)),
    ] + c31 + [
        pl.BlockSpec((1024, N), lambda j, k, nb: (k, 0)),
    ] + c32 + [
        pl.BlockSpec((N, D), lambda j, k, nb: (0, 0)),
    ] + c33
    grid_spec = pltpu.PrefetchScalarGridSpec(
        num_scalar_prefetch=1,
        grid=(nbe, nk),
        in_specs=in_specs,
        out_specs=pl.BlockSpec((_B, D), omap),
        scratch_shapes=[pltpu.VMEM((_B, N), jnp.float32),
                        pltpu.VMEM((nk, _B, 1024), jnp.bfloat16)],
    )
    return pl.pallas_call(
        body,
        grid_spec=grid_spec,
        out_shape=jax.ShapeDtypeStruct((_SLOTS, D), jnp.float32),
        input_output_aliases={1: 0},
    )(nbf, ys_prev, xs, W1, *a1, W2, *a2, W3, *a3)


def _ragged_hidden_final(ys_prev, h_in, l1, l2, act, e, nbf, first):
    W1, W2 = l1['W'], l2['W']
    K, N = W1.shape
    D = W2.shape[1]
    nk = K // 1024
    nbe = jnp.maximum(nbf[e], 1)

    def xmap(j, k, nb):
        jj = jnp.where(j < nb[e], j, _NEB)
        return ((_NEB + 1) * e + jj, k) if first else (jj, k)

    def omap(j, k, nb):
        return ((_NEB + 1) * e + jnp.where(j < nb[e], j, _NEB), 0)

    def body(nb_ref, *refs):
        if ys_prev is None:
            (x_ref, w1_ref, b1_ref, g1_ref, t1_ref,
             w2_ref, b2_ref, g2_ref, t2_ref, o_ref, acc) = refs
        else:
            (_yp, x_ref, w1_ref, b1_ref, g1_ref, t1_ref,
             w2_ref, b2_ref, g2_ref, t2_ref, o_ref, acc) = refs
        k = pl.program_id(1)
        xv = x_ref[...]
        if first:
            xv = xv.astype(jnp.bfloat16)
        prod = jnp.dot(xv, w1_ref[...].astype(jnp.bfloat16),
                       preferred_element_type=jnp.float32)

        @pl.when(k == 0)
        def _():
            acc[...] = prod

        @pl.when(k > 0)
        def _():
            acc[...] += prod

        @pl.when(k == nk - 1)
        def _():
            hh = _act_ln(act, acc[...], b1_ref[...], g1_ref[...], t1_ref[...])
            h2 = jnp.dot(hh.astype(jnp.bfloat16),
                         w2_ref[...].astype(jnp.bfloat16),
                         preferred_element_type=jnp.float32) + b2_ref[...]
            o_ref[...] = _layernorm(h2, g2_ref[...], t2_ref[...])

    c31, a1 = _c3(l1, N)
    c32, a2 = _c3(l2, D)
    in_specs = [
        pl.BlockSpec((_B, 1024), xmap),
        pl.BlockSpec((1024, N), lambda j, k, nb: (k, 0)),
    ] + c31 + [
        pl.BlockSpec((N, D), lambda j, k, nb: (0, 0)),
    ] + c32
    args = [nbf, h_in, W1] + a1 + [W2] + a2
    aliases = {}
    if ys_prev is not None:
        in_specs.insert(0, pl.BlockSpec(memory_space=pl.ANY))
        args.insert(1, ys_prev)
        aliases = {1: 0}
    grid_spec = pltpu.PrefetchScalarGridSpec(
        num_scalar_prefetch=1,
        grid=(nbe, nk),
        in_specs=in_specs,
        out_specs=pl.BlockSpec((_B, D), omap),
        scratch_shapes=[pltpu.VMEM((_B, N), jnp.float32)],
    )
    return pl.pallas_call(
        body,
        grid_spec=grid_spec,
        out_shape=jax.ShapeDtypeStruct((_SLOTS, D), jnp.float32),
        input_output_aliases=aliases,
    )(*args)


def _mix_body(v_ref, a_ref, b_ref, o_ref):
    v = v_ref[...]
    o_ref[...] = v[:, 0:1] * a_ref[...] + v[:, 1:2] * b_ref[...]


def _mix(val2, g0, g1):
    n, d = g0.shape
    bt = 512
    return pl.pallas_call(
        _mix_body,
        grid=(n // bt,),
        in_specs=[
            pl.BlockSpec((bt, 2), lambda t: (t, 0)),
            pl.BlockSpec((bt, d), lambda t: (t, 0)),
            pl.BlockSpec((bt, d), lambda t: (t, 0)),
        ],
        out_specs=pl.BlockSpec((bt, d), lambda t: (t, 0)),
        out_shape=jax.ShapeDtypeStruct((n, d), jnp.float32),
    )(val2, g0, g1)


def kernel(x, params):
    n = x.shape[0]
    val2, pos, nb8 = _gate_meta(x, params['gate_W'], params['gate_b'])
    p0 = pos[:n].reshape(n)
    p1 = pos[n:].reshape(n)
    nbf = nb8.reshape(_NE)
    xs = _sc_dispatch(x, p0, p1)
    ys = None
    for e in range(_NE):
        act, depth, _hid = _cfg(e)
        L = params['experts'][e]
        if depth == 1:
            ys = _ragged_hidden_final(ys, xs, L[0], L[1], act, e, nbf,
                                      first=True)
        elif depth == 2:
            ys = _ragged_b_full(ys, xs, L[0], L[1], L[2], act, e, nbf)
        else:
            h = _ragged_c12(xs, L[0], L[1], act, e, nbf)
            ys = _ragged_hidden_final(ys, h, L[2], L[3], act, e, nbf,
                                      first=False)
    g0, g1 = _sc_combine(ys, p0, p1)
    return _mix(val2, g0, g1)


# speedup vs baseline: 1.2098x; 1.0042x over previous
import functools

import jax
import jax.numpy as jnp
from jax import lax
from jax.experimental import pallas as pl
from jax.experimental.pallas import tpu as pltpu
from jax.experimental.pallas import tpu_sc as plsc

_ACTS = ['gelu', 'silu', 'relu', 'leaky_relu']
_DEPTHS = [1, 2, 3]
_HIDS = [1024, 2048, 3072]

_B = 256
_NEB = 8
_ECAP = (_NEB + 1) * _B
_NE = 8
_SLOTS = _NE * _ECAP


def _cfg(i):
    return _ACTS[i % 4], _DEPTHS[i % 3], _HIDS[i % 3]


def _apply_act(name, h):
    if name == 'gelu':
        return 0.5 * h * (1.0 + jax.lax.erf(h * (2.0 ** -0.5)))
    if name == 'silu':
        return h * (1.0 / (1.0 + jnp.exp(-h)))
    if name == 'relu':
        return jnp.maximum(h, 0.0)
    return jnp.where(h >= 0, h, 0.01 * h)


def _layernorm(h, g, b):
    m = jnp.mean(h, axis=-1, keepdims=True)
    v = jnp.mean((h - m) ** 2, axis=-1, keepdims=True)
    return (h - m) / jnp.sqrt(v + 1e-5) * g + b


def _act_ln(act, acc, b, g, t):
    hh = _apply_act(act, acc + b)
    return _layernorm(hh, g, t)


def _c3(v, n):
    return [pl.BlockSpec((1, n), lambda j, k, nb: (0, 0))] * 3, \
           [v['b'].reshape(1, n), v['g'].reshape(1, n), v['beta'].reshape(1, n)]


def _gate_meta_body(x_ref, gw_ref, gb_ref, v_ref, pos_ref, nb_ref):
    logits = jnp.dot(x_ref[...], gw_ref[...],
                     preferred_element_type=jnp.float32) + gb_ref[...]
    m = jnp.max(logits, axis=-1, keepdims=True)
    ex = jnp.exp(logits - m)
    s = ex / jnp.sum(ex, axis=-1, keepdims=True)
    n, e = s.shape
    col = jax.lax.broadcasted_iota(jnp.int32, (n, e), 1)
    v1 = jnp.max(s, axis=-1, keepdims=True)
    i1 = jnp.min(jnp.where(s == v1, col, e), axis=-1, keepdims=True)
    s2 = jnp.where(col == i1, -1.0, s)
    v2 = jnp.max(s2, axis=-1, keepdims=True)
    i2 = jnp.min(jnp.where(s2 == v2, col, e), axis=-1, keepdims=True)
    v_ref[...] = jnp.concatenate([v1, v2], axis=1)

    ch = 1024
    iota8 = jax.lax.broadcasted_iota(jnp.int32, (1, _NE), 1)
    rr = jax.lax.broadcasted_iota(jnp.int32, (ch, ch), 0)
    cc = jax.lax.broadcasted_iota(jnp.int32, (ch, ch), 1)
    tril = (cc < rr).astype(jnp.float32)
    base8 = (iota8 * _ECAP).astype(jnp.float32)
    carry = jnp.zeros((1, _NE), jnp.float32)
    nch = (2 * n) // ch
    for c in range(nch):
        src = i1 if c < nch // 2 else i2
        lo = (c % (nch // 2)) * ch
        ev = src[lo:lo + ch, :]
        oh = (ev == iota8).astype(jnp.float32)
        ranks = jnp.dot(tril, oh, preferred_element_type=jnp.float32) + carry
        posv = jnp.sum(oh * (ranks + base8), axis=1, keepdims=True)
        pos_ref[pl.ds(c * ch, ch), :] = posv.astype(jnp.int32)
        carry = carry + jnp.sum(oh, axis=0, keepdims=True)
    nb_ref[...] = jnp.floor((carry + (_B - 1)) * (1.0 / _B)).astype(jnp.int32)


def _gate_meta(x, gw, gb):
    n = x.shape[0]
    ne = gw.shape[1]
    return pl.pallas_call(
        _gate_meta_body,
        out_shape=[jax.ShapeDtypeStruct((n, 2), jnp.float32),
                   jax.ShapeDtypeStruct((2 * n, 1), jnp.int32),
                   jax.ShapeDtypeStruct((1, _NE), jnp.int32)],
    )(x, gw, gb.reshape(1, ne))


def _sc_dispatch(x, p0, p1):
    n, d = x.shape
    cpt = n // 32
    mesh = plsc.VectorSubcoreMesh(core_axis_name="c", subcore_axis_name="s")

    @functools.partial(
        pl.kernel, mesh=mesh,
        out_type=jax.ShapeDtypeStruct((_SLOTS, d), jnp.float32),
        scratch_types=[
            pltpu.VMEM((cpt,), jnp.int32),
            pltpu.VMEM((cpt,), jnp.int32),
            pltpu.VMEM((cpt, d), jnp.float32),
            pltpu.SemaphoreType.DMA,
            pltpu.SemaphoreType.DMA,
        ],
    )
    def k(x_hbm, p0_hbm, p1_hbm, xs_hbm, i0_v, i1_v, rows_v, sem0, sem1):
        wid = lax.axis_index("s") * 2 + lax.axis_index("c")
        base = wid * cpt
        pltpu.sync_copy(p0_hbm.at[pl.ds(base, cpt)], i0_v)
        pltpu.sync_copy(p1_hbm.at[pl.ds(base, cpt)], i1_v)
        pltpu.sync_copy(x_hbm.at[pl.ds(base, cpt)], rows_v)
        c0 = pltpu.async_copy(rows_v, xs_hbm.at[i0_v], sem0)
        c1 = pltpu.async_copy(rows_v, xs_hbm.at[i1_v], sem1)
        c0.wait()
        c1.wait()

    return k(x, p0, p1)


def _sc_combine(ys, p0, p1):
    n = p0.shape[0]
    d = ys.shape[1]
    cpt = n // 32
    half = cpt // 2
    mesh = plsc.VectorSubcoreMesh(core_axis_name="c", subcore_axis_name="s")

    @functools.partial(
        pl.kernel, mesh=mesh,
        out_type=(jax.ShapeDtypeStruct((n, d), jnp.float32),
                  jax.ShapeDtypeStruct((n, d), jnp.float32)),
        scratch_types=[
            pltpu.VMEM((half,), jnp.int32),
            pltpu.VMEM((half,), jnp.int32),
            pltpu.VMEM((half, d), jnp.float32),
            pltpu.VMEM((half, d), jnp.float32),
            pltpu.SemaphoreType.DMA,
            pltpu.SemaphoreType.DMA,
        ],
    )
    def k(ys_hbm, p0_hbm, p1_hbm, g0_hbm, g1_hbm, i0_v, i1_v,
          buf0_v, buf1_v, sem0, sem1):
        wid = lax.axis_index("s") * 2 + lax.axis_index("c")
        base = wid * cpt
        for c in range(2):
            b2 = base + c * half
            pltpu.sync_copy(p0_hbm.at[pl.ds(b2, half)], i0_v)
            pltpu.sync_copy(p1_hbm.at[pl.ds(b2, half)], i1_v)
            c0 = pltpu.async_copy(ys_hbm.at[i0_v], buf0_v, sem0)
            c1 = pltpu.async_copy(ys_hbm.at[i1_v], buf1_v, sem1)
            c0.wait()
            pltpu.sync_copy(buf0_v, g0_hbm.at[pl.ds(b2, half)])
            c1.wait()
            pltpu.sync_copy(buf1_v, g1_hbm.at[pl.ds(b2, half)])

    return k(ys, p0, p1)


def _ragged_hidden(h_in, W, b, g, beta, act, e, nbf, first):
    K, N = W.shape
    nk = K // 1024
    nbe = jnp.maximum(nbf[e], 1)

    def xmap(j, k, nb):
        jj = jnp.where(j < nb[e], j, _NEB)
        return ((_NEB + 1) * e + jj, k) if first else (jj, k)

    def omap(j, k, nb):
        return (jnp.where(j < nb[e], j, _NEB), 0)

    def body(nb_ref, x_ref, w_ref, b_ref, g_ref, t_ref, o_ref, acc):
        k = pl.program_id(1)
        xv = x_ref[...]
        if first:
            xv = xv.astype(jnp.bfloat16)
        prod = jnp.dot(xv, w_ref[...].astype(jnp.bfloat16),
                       preferred_element_type=jnp.float32)

        @pl.when(k == 0)
        def _():
            acc[...] = prod

        @pl.when(k > 0)
        def _():
            acc[...] += prod

        @pl.when(k == nk - 1)
        def _():
            o_ref[...] = _act_ln(act, acc[...], b_ref[...], g_ref[...],
                                 t_ref[...]).astype(jnp.bfloat16)

    c3, cargs = _c3({'b': b, 'g': g, 'beta': beta}, N)
    grid_spec = pltpu.PrefetchScalarGridSpec(
        num_scalar_prefetch=1,
        grid=(nbe, nk),
        in_specs=[
            pl.BlockSpec((_B, 1024), xmap),
            pl.BlockSpec((1024, N), lambda j, k, nb: (k, 0)),
        ] + c3,
        out_specs=pl.BlockSpec((_B, N), omap),
        scratch_shapes=[pltpu.VMEM((_B, N), jnp.float32)],
    )
    return pl.pallas_call(
        body,
        grid_spec=grid_spec,
        out_shape=jax.ShapeDtypeStruct((_ECAP, N), jnp.bfloat16),
    )(nbf, h_in, W, *cargs)


def _ragged_c12(xs, l1, l2, act, e, nbf):
    W1, W2 = l1['W'], l2['W']
    N = W1.shape[1]
    nk = N // 1024
    nbe = jnp.maximum(nbf[e], 1)

    def xmap(j, k, nb):
        return ((_NEB + 1) * e + jnp.where(j < nb[e], j, _NEB), 0)

    def omap(j, k, nb):
        return (jnp.where(j < nb[e], j, _NEB), 0)

    def body(nb_ref, x_ref, w1_ref, b1_ref, g1_ref, t1_ref,
             w2_ref, b2_ref, g2_ref, t2_ref, o_ref, acc, h1s):
        k = pl.program_id(1)

        @pl.when(k == 0)
        def _():
            h1 = jnp.dot(x_ref[...].astype(jnp.bfloat16),
                         w1_ref[...].astype(jnp.bfloat16),
                         preferred_element_type=jnp.float32)
            h1v = _act_ln(act, h1, b1_ref[...], g1_ref[...],
                          t1_ref[...]).astype(jnp.bfloat16)
            for i in range(nk):
                h1s[i] = h1v[:, i * 1024:(i + 1) * 1024]

        prod = jnp.dot(h1s[k],
                       w2_ref[...].astype(jnp.bfloat16),
                       preferred_element_type=jnp.float32)

        @pl.when(k == 0)
        def _():
            acc[...] = prod

        @pl.when(k > 0)
        def _():
            acc[...] += prod

        @pl.when(k == nk - 1)
        def _():
            o_ref[...] = _act_ln(act, acc[...], b2_ref[...], g2_ref[...],
                                 t2_ref[...]).astype(jnp.bfloat16)

    c31, a1 = _c3(l1, N)
    c32, a2 = _c3(l2, N)
    grid_spec = pltpu.PrefetchScalarGridSpec(
        num_scalar_prefetch=1,
        grid=(nbe, nk),
        in_specs=[
            pl.BlockSpec((_B, 1024), xmap),
            pl.BlockSpec((1024, N), lambda j, k, nb: (0, 0)),
        ] + c31 + [
            pl.BlockSpec((1024, N), lambda j, k, nb: (k, 0)),
        ] + c32,
        out_specs=pl.BlockSpec((_B, N), omap),
        scratch_shapes=[pltpu.VMEM((_B, N), jnp.float32),
                        pltpu.VMEM((nk, _B, 1024), jnp.bfloat16)],
    )
    return pl.pallas_call(
        body,
        grid_spec=grid_spec,
        out_shape=jax.ShapeDtypeStruct((_ECAP, N), jnp.bfloat16),
    )(nbf, xs, W1, *a1, W2, *a2)


def _ragged_b_full(ys_prev, xs, l1, l2, l3, act, e, nbf):
    W1, W2, W3 = l1['W'], l2['W'], l3['W']
    N = W1.shape[1]
    D = W3.shape[1]
    nk = N // 1024
    nbe = jnp.maximum(nbf[e], 1)

    def xmap(j, k, nb):
        return ((_NEB + 1) * e + jnp.where(j < nb[e], j, _NEB), 0)

    def omap(j, k, nb):
        return ((_NEB + 1) * e + jnp.where(j < nb[e], j, _NEB), 0)

    def body(nb_ref, *refs):
        (_yp, x_ref, w1_ref, b1_ref, g1_ref, t1_ref,
         w2_ref, b2_ref, g2_ref, t2_ref,
         w3_ref, b3_ref, g3_ref, t3_ref, o_ref, acc, h1s) = refs
        k = pl.program_id(1)

        @pl.when(k == 0)
        def _():
            h1 = jnp.dot(x_ref[...].astype(jnp.bfloat16),
                         w1_ref[...].astype(jnp.bfloat16),
                         preferred_element_type=jnp.float32)
            h1v = _act_ln(act, h1, b1_ref[...], g1_ref[...],
                          t1_ref[...]).astype(jnp.bfloat16)
            for i in range(nk):
                h1s[i] = h1v[:, i * 1024:(i + 1) * 1024]

        prod = jnp.dot(h1s[k],
                       w2_ref[...].astype(jnp.bfloat16),
                       preferred_element_type=jnp.float32)

        @pl.when(k == 0)
        def _():
            acc[...] = prod

        @pl.when(k > 0)
        def _():
            acc[...] += prod

        @pl.when(k == nk - 1)
        def _():
            h2 = _act_ln(act, acc[...], b2_ref[...], g2_ref[...], t2_ref[...])
            h3 = jnp.dot(h2.astype(jnp.bfloat16),
                         w3_ref[...].astype(jnp.bfloat16),
                         preferred_element_type=jnp.float32) + b3_ref[...]
            o_ref[...] = _layernorm(h3, g3_ref[...], t3_ref[...])

    c31, a1 = _c3(l1, N)
    c32, a2 = _c3(l2, N)
    c33, a3 = _c3(l3, D)
    in_specs = [
        pl.BlockSpec(memory_space=pl.ANY),
        pl.BlockSpec((_B, 1024), xmap),
        pl.BlockSpec((1024, N), lambda j, k, nb: (0, 0)),
    ] + c31 + [
        pl.BlockSpec((1024, N), lambda j, k, nb: (k, 0)),
    ] + c32 + [
        pl.BlockSpec((N, D), lambda j, k, nb: (0, 0)),
    ] + c33
    grid_spec = pltpu.PrefetchScalarGridSpec(
        num_scalar_prefetch=1,
        grid=(nbe, nk),
        in_specs=in_specs,
        out_specs=pl.BlockSpec((_B, D), omap),
        scratch_shapes=[pltpu.VMEM((_B, N), jnp.float32),
                        pltpu.VMEM((nk, _B, 1024), jnp.bfloat16)],
    )
    return pl.pallas_call(
        body,
        grid_spec=grid_spec,
        out_shape=jax.ShapeDtypeStruct((_SLOTS, D), jnp.float32),
        input_output_aliases={1: 0},
    )(nbf, ys_prev, xs, W1, *a1, W2, *a2, W3, *a3)


def _ragged_a_all(xs, ls, nbf):
    eids = (0, 3, 6)
    acts = [_cfg(i)[0] for i in eids]
    D = 1024
    ntot = jnp.maximum(nbf[0] + nbf[3] + nbf[6], 1)

    def bidx(j, nb):
        n0 = nb[0]
        n03 = nb[0] + nb[3]
        tot = n03 + nb[6]
        base = _NEB + 1
        return jnp.where(
            j < n0, j,
            jnp.where(j < n03, base * 3 + (j - n0),
                      jnp.where(j < tot, base * 6 + (j - n03),
                                base * 6 + _NEB)))

    xmap = lambda j, nb: (bidx(j, nb), 0)
    omap = lambda j, nb: (bidx(j, nb), 0)
    cmap = lambda j, nb: (0, 0)

    def body(nb_ref, x_ref, *refs):
        o_ref = refs[-1]
        wrefs = refs[:-1]
        j = pl.program_id(0)
        sel = ((j >= nb_ref[0]).astype(jnp.int32)
               + (j >= nb_ref[0] + nb_ref[3]).astype(jnp.int32))
        xv = x_ref[...].astype(jnp.bfloat16)

        def mk(i):
            w1, b1, g1, t1, w2, b2, g2, t2 = wrefs[8 * i:8 * i + 8]

            def br():
                h1 = jnp.dot(xv, w1[...].astype(jnp.bfloat16),
                             preferred_element_type=jnp.float32)
                h1 = _act_ln(acts[i], h1, b1[...], g1[...], t1[...])
                h2 = jnp.dot(h1.astype(jnp.bfloat16),
                             w2[...].astype(jnp.bfloat16),
                             preferred_element_type=jnp.float32) + b2[...]
                return _layernorm(h2, g2[...], t2[...])

            return br

        o_ref[...] = lax.switch(sel, [mk(0), mk(1), mk(2)])

    in_specs = [pl.BlockSpec((_B, 1024), xmap)]
    args = [nbf, xs]
    cs = [pl.BlockSpec((1, D), cmap)] * 3
    for l1, l2 in ls:
        _u1, a1 = _c3(l1, D)
        _u2, a2 = _c3(l2, D)
        in_specs += [pl.BlockSpec((1024, D), cmap)] + cs \
            + [pl.BlockSpec((D, D), cmap)] + cs
        args += [l1['W']] + a1 + [l2['W']] + a2
    grid_spec = pltpu.PrefetchScalarGridSpec(
        num_scalar_prefetch=1,
        grid=(ntot,),
        in_specs=in_specs,
        out_specs=pl.BlockSpec((_B, D), omap),
    )
    return pl.pallas_call(
        body,
        grid_spec=grid_spec,
        out_shape=jax.ShapeDtypeStruct((_SLOTS, D), jnp.float32),
    )(*args)


def _ragged_hidden_final(ys_prev, h_in, l1, l2, act, e, nbf, first):
    W1, W2 = l1['W'], l2['W']
    K, N = W1.shape
    D = W2.shape[1]
    nk = K // 1024
    nbe = jnp.maximum(nbf[e], 1)

    def xmap(j, k, nb):
        jj = jnp.where(j < nb[e], j, _NEB)
        return ((_NEB + 1) * e + jj, k) if first else (jj, k)

    def omap(j, k, nb):
        return ((_NEB + 1) * e + jnp.where(j < nb[e], j, _NEB), 0)

    def body(nb_ref, *refs):
        if ys_prev is None:
            (x_ref, w1_ref, b1_ref, g1_ref, t1_ref,
             w2_ref, b2_ref, g2_ref, t2_ref, o_ref, acc) = refs
        else:
            (_yp, x_ref, w1_ref, b1_ref, g1_ref, t1_ref,
             w2_ref, b2_ref, g2_ref, t2_ref, o_ref, acc) = refs
        k = pl.program_id(1)
        xv = x_ref[...]
        if first:
            xv = xv.astype(jnp.bfloat16)
        prod = jnp.dot(xv, w1_ref[...].astype(jnp.bfloat16),
                       preferred_element_type=jnp.float32)

        @pl.when(k == 0)
        def _():
            acc[...] = prod

        @pl.when(k > 0)
        def _():
            acc[...] += prod

        @pl.when(k == nk - 1)
        def _():
            hh = _act_ln(act, acc[...], b1_ref[...], g1_ref[...], t1_ref[...])
            h2 = jnp.dot(hh.astype(jnp.bfloat16),
                         w2_ref[...].astype(jnp.bfloat16),
                         preferred_element_type=jnp.float32) + b2_ref[...]
            o_ref[...] = _layernorm(h2, g2_ref[...], t2_ref[...])

    c31, a1 = _c3(l1, N)
    c32, a2 = _c3(l2, D)
    in_specs = [
        pl.BlockSpec((_B, 1024), xmap),
        pl.BlockSpec((1024, N), lambda j, k, nb: (k, 0)),
    ] + c31 + [
        pl.BlockSpec((N, D), lambda j, k, nb: (0, 0)),
    ] + c32
    args = [nbf, h_in, W1] + a1 + [W2] + a2
    aliases = {}
    if ys_prev is not None:
        in_specs.insert(0, pl.BlockSpec(memory_space=pl.ANY))
        args.insert(1, ys_prev)
        aliases = {1: 0}
    grid_spec = pltpu.PrefetchScalarGridSpec(
        num_scalar_prefetch=1,
        grid=(nbe, nk),
        in_specs=in_specs,
        out_specs=pl.BlockSpec((_B, D), omap),
        scratch_shapes=[pltpu.VMEM((_B, N), jnp.float32)],
    )
    return pl.pallas_call(
        body,
        grid_spec=grid_spec,
        out_shape=jax.ShapeDtypeStruct((_SLOTS, D), jnp.float32),
        input_output_aliases=aliases,
    )(*args)


def _mix_body(v_ref, a_ref, b_ref, o_ref):
    v = v_ref[...]
    o_ref[...] = v[:, 0:1] * a_ref[...] + v[:, 1:2] * b_ref[...]


def _mix(val2, g0, g1):
    n, d = g0.shape
    bt = 512
    return pl.pallas_call(
        _mix_body,
        grid=(n // bt,),
        in_specs=[
            pl.BlockSpec((bt, 2), lambda t: (t, 0)),
            pl.BlockSpec((bt, d), lambda t: (t, 0)),
            pl.BlockSpec((bt, d), lambda t: (t, 0)),
        ],
        out_specs=pl.BlockSpec((bt, d), lambda t: (t, 0)),
        out_shape=jax.ShapeDtypeStruct((n, d), jnp.float32),
    )(val2, g0, g1)


def kernel(x, params):
    n = x.shape[0]
    val2, pos, nb8 = _gate_meta(x, params['gate_W'], params['gate_b'])
    p0 = pos[:n].reshape(n)
    p1 = pos[n:].reshape(n)
    nbf = nb8.reshape(_NE)
    xs = _sc_dispatch(x, p0, p1)
    ex = params['experts']
    ys = _ragged_a_all(xs, [(ex[i][0], ex[i][1]) for i in (0, 3, 6)], nbf)
    for e in (1, 4, 7):
        act, _d, _h = _cfg(e)
        L = ex[e]
        ys = _ragged_b_full(ys, xs, L[0], L[1], L[2], act, e, nbf)
    for e in (2, 5):
        act, _d, _h = _cfg(e)
        L = ex[e]
        h = _ragged_c12(xs, L[0], L[1], act, e, nbf)
        ys = _ragged_hidden_final(ys, h, L[2], L[3], act, e, nbf,
                                  first=False)
    g0, g1 = _sc_combine(ys, p0, p1)
    return _mix(val2, g0, g1)
